# Initial kernel scaffold; baseline (speedup 1.0000x reference)
import jax
import jax.numpy as jnp
from jax.experimental import pallas as pl


def kernel(x, edge_index, Wl0, Wr0, b0, Wl1, Wr1, b1, Wl2, Wr2, b2, Wl3, Wr3, b3, Wl4, Wr4, b4, Wl5, Wr5, b5):
    raise NotImplementedError("write your pallas kernel here")


# speedup vs baseline: 4.9919x; 4.9919x over previous
import functools

import jax
import jax.numpy as jnp
from jax import lax
from jax.experimental import pallas as pl
from jax.experimental.pallas import tpu as pltpu
from jax.experimental.pallas import tpu_sc as plsc

N = 10000
E = 320000
W = 128
NC = 2
NS = 16
NW = NC * NS
TPE = E // NW
B = 80
NB = TPE // B
NPAD = 10240
NR = NPAD // NS
CH = 128
NCH = NR // CH
BR = 1000
EB = 2000
QROWS = NPAD // W


def _sc_agg(p, e_src, e_dst):
    mesh = plsc.VectorSubcoreMesh(core_axis_name="c", subcore_axis_name="s")

    @functools.partial(
        pl.kernel,
        out_type=jax.ShapeDtypeStruct((NC, NPAD, W), jnp.float32),
        mesh=mesh,
        scratch_types=[
            pltpu.VMEM((B,), jnp.int32),
            pltpu.VMEM((B,), jnp.int32),
            pltpu.VMEM((B, W), jnp.float32),
            pltpu.VMEM((CH, W), jnp.float32),
            pltpu.VMEM_SHARED((NPAD, W), jnp.float32),
            pltpu.SemaphoreType.DMA,
        ],
    )
    def sc_agg(p_hbm, src_hbm, dst_hbm, out_hbm, idx_s, idx_d, rows, zbuf, acc, sem):
        cid = lax.axis_index("c")
        sid = lax.axis_index("s")
        wid = cid * NS + sid
        zero16 = jnp.zeros((16,), jnp.float32)

        def zrow(i, carry):
            def zcol(j, c2):
                zbuf[i, pl.ds(j * 16, 16)] = zero16
                return c2
            return lax.fori_loop(0, W // 16, zcol, carry)

        lax.fori_loop(0, CH, zrow, 0)
        row0 = sid * NR
        for c in range(NCH):
            pltpu.sync_copy(zbuf, acc.at[pl.ds(row0 + c * CH, CH)])
        plsc.subcore_barrier()

        def step(b, carry):
            base = wid * TPE + b * B
            pltpu.sync_copy(src_hbm.at[pl.ds(base, B)], idx_s)
            pltpu.sync_copy(dst_hbm.at[pl.ds(base, B)], idx_d)
            pltpu.async_copy(p_hbm.at[idx_s], rows, sem).wait()
            pltpu.sync_copy(rows, acc.at[idx_d], add=True)
            return carry

        lax.fori_loop(0, NB, step, 0)
        plsc.subcore_barrier()
        for c in range(NCH):
            pltpu.sync_copy(acc.at[pl.ds(row0 + c * CH, CH)], zbuf)
            pltpu.sync_copy(zbuf, out_hbm.at[cid, pl.ds(row0 + c * CH, CH)])

    return sc_agg(p, e_src, e_dst)


def _tc_deg(dst3):
    def body(d_ref, c_ref):
        i = pl.program_id(0)

        @pl.when(i == 0)
        def _():
            c_ref[...] = jnp.zeros((QROWS, W), jnp.float32)

        d = d_ref[0, 0, :]
        q = d // W
        r = d - q * W
        u = (q[:, None] == lax.broadcasted_iota(jnp.int32, (EB, QROWS), 1)
             ).astype(jnp.float32)
        v = (r[:, None] == lax.broadcasted_iota(jnp.int32, (EB, W), 1)
             ).astype(jnp.float32)
        c_ref[...] += lax.dot_general(
            u, v, (((0,), (0,)), ((), ())), preferred_element_type=jnp.float32)

    return pl.pallas_call(
        body,
        grid=(E // EB,),
        in_specs=[pl.BlockSpec((1, 1, EB), lambda i: (i, 0, 0))],
        out_specs=pl.BlockSpec((QROWS, W), lambda i: (0, 0)),
        out_shape=jax.ShapeDtypeStruct((QROWS, W), jnp.float32),
    )(dst3)


def _tc_first(x, wlT, wrT, b):
    def body(x_ref, wl_ref, wr_ref, b_ref, p_ref, r_ref):
        h = x_ref[...]
        p_ref[...] = jnp.dot(h, wl_ref[...], preferred_element_type=jnp.float32)
        r_ref[...] = jnp.dot(h, wr_ref[...], preferred_element_type=jnp.float32) + b_ref[...]

    return pl.pallas_call(
        body,
        grid=(N // BR,),
        in_specs=[
            pl.BlockSpec((BR, 128), lambda i: (i, 0)),
            pl.BlockSpec((128, 128), lambda i: (0, 0)),
            pl.BlockSpec((128, 128), lambda i: (0, 0)),
            pl.BlockSpec((1, 128), lambda i: (0, 0)),
        ],
        out_specs=[
            pl.BlockSpec((BR, 128), lambda i: (i, 0)),
            pl.BlockSpec((BR, 128), lambda i: (i, 0)),
        ],
        out_shape=[
            jax.ShapeDtypeStruct((N, 128), jnp.float32),
            jax.ShapeDtypeStruct((N, 128), jnp.float32),
        ],
    )(x, wlT, wrT, b)


def _tc_mid(s, deg, r, wlT, wrT, b, act):
    def body(s_ref, deg_ref, r_ref, wl_ref, wr_ref, b_ref, p_ref, ro_ref):
        inv = 1.0 / jnp.maximum(deg_ref[...], 1.0)
        h = (s_ref[0] + s_ref[1]) * inv + r_ref[...]
        if act:
            h = jnp.where(h > 0, h, 0.01 * h)
        p_ref[...] = jnp.dot(h, wl_ref[...], preferred_element_type=jnp.float32)
        ro_ref[...] = jnp.dot(h, wr_ref[...], preferred_element_type=jnp.float32) + b_ref[...]

    return pl.pallas_call(
        body,
        grid=(N // BR,),
        in_specs=[
            pl.BlockSpec((NC, BR, 128), lambda i: (0, i, 0)),
            pl.BlockSpec((BR, 1), lambda i: (i, 0)),
            pl.BlockSpec((BR, 128), lambda i: (i, 0)),
            pl.BlockSpec((128, 128), lambda i: (0, 0)),
            pl.BlockSpec((128, 128), lambda i: (0, 0)),
            pl.BlockSpec((1, 128), lambda i: (0, 0)),
        ],
        out_specs=[
            pl.BlockSpec((BR, 128), lambda i: (i, 0)),
            pl.BlockSpec((BR, 128), lambda i: (i, 0)),
        ],
        out_shape=[
            jax.ShapeDtypeStruct((N, 128), jnp.float32),
            jax.ShapeDtypeStruct((N, 128), jnp.float32),
        ],
    )(s, deg, r, wlT, wrT, b)


def _tc_last(s, deg, r):
    def body(s_ref, deg_ref, r_ref, o_ref):
        inv = 1.0 / jnp.maximum(deg_ref[...], 1.0)
        o_ref[...] = (s_ref[0] + s_ref[1]) * inv + r_ref[...]

    return pl.pallas_call(
        body,
        grid=(N // BR,),
        in_specs=[
            pl.BlockSpec((NC, BR, 128), lambda i: (0, i, 0)),
            pl.BlockSpec((BR, 1), lambda i: (i, 0)),
            pl.BlockSpec((BR, 128), lambda i: (i, 0)),
        ],
        out_specs=pl.BlockSpec((BR, 128), lambda i: (i, 0)),
        out_shape=jax.ShapeDtypeStruct((N, 128), jnp.float32),
    )(s, deg, r)


def kernel(x, edge_index, Wl0, Wr0, b0, Wl1, Wr1, b1, Wl2, Wr2, b2,
           Wl3, Wr3, b3, Wl4, Wr4, b4, Wl5, Wr5, b5):
    e_src = edge_index[0]
    e_dst = edge_index[1]
    dst3 = e_dst.reshape(E // EB, 1, EB)

    deg2d = _tc_deg(dst3)
    deg = deg2d.reshape(NPAD)[:N].reshape(N, 1)

    p0, r0 = _tc_first(x, Wl0.T, Wr0.T, b0.reshape(1, -1))
    s0 = _sc_agg(p0, e_src, e_dst)[:, :N]
    p1, r1 = _tc_mid(s0, deg, r0, Wl1.T, Wr1.T, b1.reshape(1, -1), act=True)
    s1 = _sc_agg(p1, e_src, e_dst)[:, :N]
    p2, r2 = _tc_mid(s1, deg, r1, Wl2.T, Wr2.T, b2.reshape(1, -1), act=True)
    s2 = _sc_agg(p2, e_src, e_dst)[:, :N]
    p3, r3 = _tc_mid(s2, deg, r2, Wl3.T, Wr3.T, b3.reshape(1, -1), act=True)
    s3 = _sc_agg(p3, e_src, e_dst)[:, :N]
    wl45T = jnp.concatenate([Wl4, Wl5], axis=0).T
    wr45T = jnp.concatenate([Wr4, Wr5], axis=0).T
    b45 = jnp.concatenate([b4, b5]).reshape(1, -1)
    p45, r45 = _tc_mid(s3, deg, r3, wl45T, wr45T, b45, act=False)
    s45 = _sc_agg(p45, e_src, e_dst)[:, :N]
    o = _tc_last(s45, deg, r45)
    return o[:, :64], o[:, 64:]
